# Initial kernel scaffold; baseline (speedup 1.0000x reference)
import jax
import jax.numpy as jnp
from jax.experimental import pallas as pl


def kernel(fatoms, fbonds, agraph, bgraph, W_i, W_h, W_o, b_o, W_mh, b_mh, W_mo, b_mo):
    raise NotImplementedError("write your pallas kernel here")


# speedup vs baseline: 3.5099x; 3.5099x over previous
import functools

import jax
import jax.numpy as jnp
from jax import lax
from jax.experimental import pallas as pl
from jax.experimental.pallas import tpu as pltpu
from jax.experimental.pallas import tpu_sc as plsc

H = 128
NEI = 6
NC, NS = 2, 16
NW = NC * NS
G = 120
LG = H // 16


def _make_gather_sum(n_src, n_dst, chunk):
    assert n_dst % (NW * chunk) == 0
    assert (chunk * NEI) % G == 0
    per_w = n_dst // NW
    n_chunks = per_w // chunk
    R = (chunk * NEI) // G
    rows_per_w = per_w * NEI // G

    def body(table_hbm, idx_hbm, out_hbm, idx_v, rows_v, acc_v, sem):
        wid = lax.axis_index("s") * NC + lax.axis_index("c")
        out_base = wid * per_w
        pltpu.sync_copy(idx_hbm.at[wid], idx_v)

        def chunk_body(ci, carry):
            cps = [
                pltpu.async_copy(
                    table_hbm.at[idx_v.at[ci * R + g]],
                    rows_v.at[pl.ds(g * G, G)],
                    sem,
                )
                for g in range(R)
            ]
            for cp in cps:
                cp.wait()

            def bond_body(j, c2):
                base = j * NEI
                for l in range(LG):
                    s = pl.ds(l * 16, 16)
                    a = rows_v[base, s]
                    for k in range(1, NEI):
                        a = a + rows_v[base + k, s]
                    acc_v[j, s] = a
                return c2

            lax.fori_loop(0, chunk, bond_body, 0)
            pltpu.sync_copy(acc_v, out_hbm.at[pl.ds(out_base + ci * chunk, chunk)])
            return carry

        lax.fori_loop(0, n_chunks, chunk_body, 0)

    return pl.kernel(
        body,
        out_type=jax.ShapeDtypeStruct((n_dst, H), jnp.float32),
        mesh=plsc.VectorSubcoreMesh(
            core_axis_name="c", subcore_axis_name="s",
            num_cores=NC, num_subcores=NS),
        scratch_types=[
            pltpu.VMEM((rows_per_w, G), jnp.int32),
            pltpu.VMEM((chunk * NEI, H), jnp.float32),
            pltpu.VMEM((chunk, H), jnp.float32),
            pltpu.SemaphoreType.DMA,
        ],
        name=f"sc_gather_sum_{n_dst}",
    )


def _tc_input_proj(x, w, bm):
    m, k = x.shape
    n = w.shape[1]

    def body(x_ref, w_ref, pre_ref, msg_ref):
        acc = jnp.dot(x_ref[...], w_ref[...], preferred_element_type=jnp.float32)
        pre_ref[...] = acc
        msg_ref[...] = jnp.maximum(acc, 0.0)

    return pl.pallas_call(
        body,
        grid=(m // bm,),
        in_specs=[
            pl.BlockSpec((bm, k), lambda i: (i, 0)),
            pl.BlockSpec((k, n), lambda i: (0, 0)),
        ],
        out_specs=[
            pl.BlockSpec((bm, n), lambda i: (i, 0)),
            pl.BlockSpec((bm, n), lambda i: (i, 0)),
        ],
        out_shape=[
            jax.ShapeDtypeStruct((m, n), jnp.float32),
            jax.ShapeDtypeStruct((m, n), jnp.float32),
        ],
        name="tc_input_proj",
    )(x, w)


def _tc_layer_update(s, pre, w, bm):
    m, k = s.shape
    n = w.shape[1]

    def body(s_ref, pre_ref, w_ref, out_ref):
        acc = jnp.dot(s_ref[...], w_ref[...], preferred_element_type=jnp.float32)
        out_ref[...] = jnp.maximum(pre_ref[...] + acc, 0.0)

    return pl.pallas_call(
        body,
        grid=(m // bm,),
        in_specs=[
            pl.BlockSpec((bm, k), lambda i: (i, 0)),
            pl.BlockSpec((bm, n), lambda i: (i, 0)),
            pl.BlockSpec((k, n), lambda i: (0, 0)),
        ],
        out_specs=pl.BlockSpec((bm, n), lambda i: (i, 0)),
        out_shape=jax.ShapeDtypeStruct((m, n), jnp.float32),
        name="tc_layer_update",
    )(s, pre, w)


def _tc_readout(fa, na, w_a, w_n, b, bm):
    m = fa.shape[0]
    n = w_a.shape[1]

    def body(fa_ref, na_ref, wa_ref, wn_ref, b_ref, out_ref):
        acc = jnp.dot(fa_ref[...], wa_ref[...], preferred_element_type=jnp.float32)
        acc = acc + jnp.dot(na_ref[...], wn_ref[...], preferred_element_type=jnp.float32)
        out_ref[...] = acc + b_ref[...]

    return pl.pallas_call(
        body,
        grid=(m // bm,),
        in_specs=[
            pl.BlockSpec((bm, fa.shape[1]), lambda i: (i, 0)),
            pl.BlockSpec((bm, na.shape[1]), lambda i: (i, 0)),
            pl.BlockSpec((fa.shape[1], n), lambda i: (0, 0)),
            pl.BlockSpec((na.shape[1], n), lambda i: (0, 0)),
            pl.BlockSpec((1, n), lambda i: (0, 0)),
        ],
        out_specs=pl.BlockSpec((bm, n), lambda i: (i, 0)),
        out_shape=jax.ShapeDtypeStruct((m, n), jnp.float32),
        name="tc_readout",
    )(fa, na, w_a, w_n, b)


def _tc_mol_head(atom3, w_mh, b_mh, w_mo_row, b_mo):
    n_mols, per_mol, h = atom3.shape
    ffn = w_mh.shape[1]

    def body(x_ref, wmh_ref, bmh_ref, wmo_ref, bmo_ref, out_ref):
        pooled = jnp.sum(x_ref[...], axis=1) * (1.0 / per_mol)
        hid = jnp.dot(pooled, wmh_ref[...], preferred_element_type=jnp.float32)
        hid = jnp.maximum(hid + bmh_ref[...], 0.0)
        o = jnp.sum(hid * wmo_ref[...], axis=1, keepdims=True)
        out_ref[...] = o + bmo_ref[...]

    return pl.pallas_call(
        body,
        in_specs=[
            pl.BlockSpec((n_mols, per_mol, h), lambda: (0, 0, 0)),
            pl.BlockSpec((h, ffn), lambda: (0, 0)),
            pl.BlockSpec((1, ffn), lambda: (0, 0)),
            pl.BlockSpec((1, ffn), lambda: (0, 0)),
            pl.BlockSpec((1, 1), lambda: (0, 0)),
        ],
        out_specs=pl.BlockSpec((n_mols, 1), lambda: (0, 0)),
        out_shape=jax.ShapeDtypeStruct((n_mols, 1), jnp.float32),
        name="tc_mol_head",
    )(atom3, w_mh, b_mh, w_mo_row, b_mo)


N_ATOMS = 10000
N_BONDS = 160000
N_ATOMS_PAD = 10240
N_LAYERS = 5
N_MOLS = 200
ATOMS_PER_MOL = 50

_gs_bonds = _make_gather_sum(N_BONDS, N_BONDS, chunk=40)
_gs_atoms = _make_gather_sum(N_BONDS, N_ATOMS_PAD, chunk=80)


def kernel(fatoms, fbonds, agraph, bgraph, W_i, W_h, W_o, b_o, W_mh, b_mh, W_mo, b_mo):
    bond_idx = bgraph.reshape(NW, N_BONDS * NEI // G // NW, G)
    atom_idx = jnp.pad(agraph, ((0, N_ATOMS_PAD - N_ATOMS), (0, 0))).reshape(
        NW, N_ATOMS_PAD * NEI // G // NW, G)

    pre, msg = _tc_input_proj(fbonds, W_i, bm=2000)
    for _ in range(N_LAYERS - 1):
        s = _gs_bonds(msg, bond_idx)
        msg = _tc_layer_update(s, pre, W_h, bm=2000)

    na = _gs_atoms(msg, atom_idx)[:N_ATOMS]
    atom_h = _tc_readout(fatoms, na, W_o[:H], W_o[H:], b_o.reshape(1, H), bm=1000)
    mol_o = _tc_mol_head(
        atom_h.reshape(N_MOLS, ATOMS_PER_MOL, H),
        W_mh, b_mh.reshape(1, -1), W_mo.reshape(1, -1), b_mo.reshape(1, 1))
    return (atom_h, mol_o)
